# baseline (speedup 1.0000x reference)
import functools

import jax
import jax.numpy as jnp
from jax import lax
from jax.experimental import pallas as pl
from jax.experimental.pallas import tpu as pltpu
from jax.experimental.pallas import tpu_sc as plsc

E = 320000
D = 128
N_NODES = 10000
NPAD = 10240
BLK = 16000
NS = 16
EPW = E // NS
CHUNKS = EPW // 16
SEG = NPAD // NS


def _alpha_full(h, t):
    ones1 = jnp.ones((D, 1), jnp.float32)
    ht = jnp.dot(h * t, ones1, preferred_element_type=jnp.float32)
    hh = jnp.dot(h * h, ones1, preferred_element_type=jnp.float32)
    tt = jnp.dot(t * t, ones1, preferred_element_type=jnp.float32)
    prod = hh * tt
    alpha = jnp.where(prod > 0,
                      (ht * lax.rsqrt(prod) + 1.0) * 0.5,
                      jnp.float32(0.5))
    dense = jnp.dot(alpha, jnp.ones((1, 128), jnp.float32),
                    preferred_element_type=jnp.float32)
    d3 = dense.reshape(BLK // 128, 128, 128)
    row = lax.broadcasted_iota(jnp.int32, (1, 128, 128), 1)
    col = lax.broadcasted_iota(jnp.int32, (1, 128, 128), 2)
    eye = jnp.where(row == col, jnp.float32(1.0), jnp.float32(0.0))
    return jnp.sum(d3 * eye, axis=1)


def _alpha_body(h_ref, t_ref, o_ref):
    o_ref[...] = _alpha_full(h_ref[...], t_ref[...]).reshape(
        1, BLK // 128, 128)


_alpha_tc = pl.pallas_call(
    _alpha_body,
    grid=(E // BLK,),
    in_specs=[
        pl.BlockSpec((BLK, D), lambda i: (i, 0)),
        pl.BlockSpec((BLK, D), lambda i: (i, 0)),
    ],
    out_specs=pl.BlockSpec((1, BLK // 128, 128), lambda i: (i, 0, 0)),
    out_shape=jax.ShapeDtypeStruct((E // BLK, BLK // 128, 128),
                                   jnp.float32),
)


def _sc_body(alpha_hbm, hl_hbm, tl_hbm, gidx_hbm, out_hbm,
             hl_v, al_v, tl_v, acc, parts, partials, dinv_sh):
    sid = lax.axis_index("s")
    base = sid * EPW
    pltpu.sync_copy(hl_hbm.at[pl.ds(base, EPW)], hl_v)
    pltpu.sync_copy(alpha_hbm.at[pl.ds(base, EPW)], al_v)
    pltpu.sync_copy(tl_hbm.at[pl.ds(base, EPW)], tl_v)
    pltpu.sync_copy(hl_v, gidx_hbm.at[pl.ds(base, EPW)])
    pltpu.sync_copy(tl_v, gidx_hbm.at[pl.ds(E + base, EPW)])

    zero16 = jnp.zeros((16,), jnp.float32)

    @plsc.parallel_loop(0, NPAD // 16, unroll=4)
    def _(i):
        acc[pl.ds(i * 16, 16)] = zero16

    @plsc.parallel_loop(0, CHUNKS, unroll=2)
    def _(i):
        idx = hl_v[pl.ds(i * 16, 16)]
        a = al_v[pl.ds(i * 16, 16)]
        plsc.addupdate_scatter(acc, [idx], a)

    pltpu.sync_copy(acc, partials.at[sid])
    plsc.subcore_barrier()

    seg0 = sid * SEG
    for k in range(NS):
        pltpu.sync_copy(partials.at[k, pl.ds(seg0, SEG)], parts.at[k])

    inf = jnp.float32(jnp.inf)

    @plsc.parallel_loop(0, SEG // 16, unroll=2)
    def _(c):
        s = parts[0, pl.ds(c * 16, 16)]
        for k in range(1, NS):
            s = s + parts[k, pl.ds(c * 16, 16)]
        dinv = jnp.float32(1.0) / s
        dinv = jnp.where(jnp.abs(dinv) < inf, dinv, jnp.float32(0.0))
        parts[0, pl.ds(c * 16, 16)] = dinv

    pltpu.sync_copy(parts.at[0], dinv_sh.at[pl.ds(seg0, SEG)])
    plsc.subcore_barrier()

    pltpu.sync_copy(dinv_sh, acc)

    @plsc.parallel_loop(0, CHUNKS, unroll=2)
    def _(i):
        idx = hl_v[pl.ds(i * 16, 16)]
        d = plsc.load_gather(acc, [idx])
        al_v[pl.ds(i * 16, 16)] = d * al_v[pl.ds(i * 16, 16)]

    pltpu.sync_copy(al_v, out_hbm.at[pl.ds(base, EPW)])


@functools.lru_cache(maxsize=1)
def _gvalues_sc():
    mesh = plsc.VectorSubcoreMesh(
        core_axis_name="c", subcore_axis_name="s", num_cores=1)
    return pl.kernel(
        _sc_body,
        out_type=[
            jax.ShapeDtypeStruct((2 * E,), jnp.int32),
            jax.ShapeDtypeStruct((E,), jnp.float32),
        ],
        mesh=mesh,
        compiler_params=pltpu.CompilerParams(needs_layout_passes=False),
        scratch_types=[
            pltpu.VMEM((EPW,), jnp.int32),
            pltpu.VMEM((EPW,), jnp.float32),
            pltpu.VMEM((EPW,), jnp.int32),
            pltpu.VMEM((NPAD,), jnp.float32),
            pltpu.VMEM((NS, SEG), jnp.float32),
            pltpu.VMEM_SHARED((NS, NPAD), jnp.float32),
            pltpu.VMEM_SHARED((NPAD,), jnp.float32),
        ],
    )


def kernel(head_embeds, tail_embeds, head_list, tail_list):
    out = _alpha_tc(head_embeds, tail_embeds)
    alpha = out.reshape(E)
    gidx_flat, g_values = _gvalues_sc()(alpha, head_list, tail_list)
    return (gidx_flat.reshape(2, E), g_values)


# speedup vs baseline: 1.0645x; 1.0645x over previous
import functools

import jax
import jax.numpy as jnp
from jax import lax
from jax.experimental import pallas as pl
from jax.experimental.pallas import tpu as pltpu
from jax.experimental.pallas import tpu_sc as plsc

E = 320000
D = 128
N_NODES = 10000
NPAD = 10240
BLK = 16000
NS = 16
EPW = E // NS
CHUNKS = EPW // 16
SEG = NPAD // NS


def _alpha_full(h, t):
    ones1 = jnp.ones((D, 1), jnp.float32)
    ht = jnp.dot(h * t, ones1, preferred_element_type=jnp.float32)
    hh = jnp.dot(h * h, ones1, preferred_element_type=jnp.float32)
    tt = jnp.dot(t * t, ones1, preferred_element_type=jnp.float32)
    prod = hh * tt
    alpha = jnp.where(prod > 0,
                      (ht * lax.rsqrt(prod) + 1.0) * 0.5,
                      jnp.float32(0.5))
    dense = jnp.dot(alpha, jnp.ones((1, 128), jnp.float32),
                    preferred_element_type=jnp.float32)
    d3 = dense.reshape(BLK // 128, 128, 128)
    row = lax.broadcasted_iota(jnp.int32, (1, 128, 128), 1)
    col = lax.broadcasted_iota(jnp.int32, (1, 128, 128), 2)
    eye = jnp.where(row == col, jnp.float32(1.0), jnp.float32(0.0))
    return jnp.sum(d3 * eye, axis=1)


def _alpha_body(h_ref, t_ref, o_ref):
    o_ref[...] = _alpha_full(h_ref[...], t_ref[...]).reshape(
        1, BLK // 128, 128)


_alpha_tc = pl.pallas_call(
    _alpha_body,
    grid=(E // BLK,),
    in_specs=[
        pl.BlockSpec((BLK, D), lambda i: (i, 0)),
        pl.BlockSpec((BLK, D), lambda i: (i, 0)),
    ],
    out_specs=pl.BlockSpec((1, BLK // 128, 128), lambda i: (i, 0, 0)),
    out_shape=jax.ShapeDtypeStruct((E // BLK, BLK // 128, 128),
                                   jnp.float32),
)


def _sc_body(alpha_hbm, hl_hbm, out_hbm,
             hl_v, al_v, acc, parts, partials, dinv_sh):
    sid = lax.axis_index("s")
    base = sid * EPW
    pltpu.sync_copy(hl_hbm.at[pl.ds(base, EPW)], hl_v)
    pltpu.sync_copy(alpha_hbm.at[pl.ds(base, EPW)], al_v)

    zero16 = jnp.zeros((16,), jnp.float32)

    @plsc.parallel_loop(0, NPAD // 16, unroll=4)
    def _(i):
        acc[pl.ds(i * 16, 16)] = zero16

    @plsc.parallel_loop(0, CHUNKS, unroll=10)
    def _(i):
        idx = hl_v[pl.ds(i * 16, 16)]
        a = al_v[pl.ds(i * 16, 16)]
        plsc.addupdate_scatter(acc, [idx], a)

    pltpu.sync_copy(acc, partials.at[sid])
    plsc.subcore_barrier()

    seg0 = sid * SEG
    for k in range(NS):
        pltpu.sync_copy(partials.at[k, pl.ds(seg0, SEG)], parts.at[k])

    inf = jnp.float32(jnp.inf)

    @plsc.parallel_loop(0, SEG // 16, unroll=2)
    def _(c):
        s = parts[0, pl.ds(c * 16, 16)]
        for k in range(1, NS):
            s = s + parts[k, pl.ds(c * 16, 16)]
        dinv = jnp.float32(1.0) / s
        dinv = jnp.where(jnp.abs(dinv) < inf, dinv, jnp.float32(0.0))
        parts[0, pl.ds(c * 16, 16)] = dinv

    pltpu.sync_copy(parts.at[0], dinv_sh.at[pl.ds(seg0, SEG)])
    plsc.subcore_barrier()

    pltpu.sync_copy(dinv_sh, acc)

    @plsc.parallel_loop(0, CHUNKS, unroll=10)
    def _(i):
        idx = hl_v[pl.ds(i * 16, 16)]
        d = plsc.load_gather(acc, [idx])
        al_v[pl.ds(i * 16, 16)] = d * al_v[pl.ds(i * 16, 16)]

    pltpu.sync_copy(al_v, out_hbm.at[pl.ds(base, EPW)])


@functools.lru_cache(maxsize=1)
def _gvalues_sc():
    mesh = plsc.VectorSubcoreMesh(
        core_axis_name="c", subcore_axis_name="s", num_cores=1)
    return pl.kernel(
        _sc_body,
        out_type=jax.ShapeDtypeStruct((E,), jnp.float32),
        mesh=mesh,
        compiler_params=pltpu.CompilerParams(needs_layout_passes=False),
        scratch_types=[
            pltpu.VMEM((EPW,), jnp.int32),
            pltpu.VMEM((EPW,), jnp.float32),
            pltpu.VMEM((NPAD,), jnp.float32),
            pltpu.VMEM((NS, SEG), jnp.float32),
            pltpu.VMEM_SHARED((NS, NPAD), jnp.float32),
            pltpu.VMEM_SHARED((NPAD,), jnp.float32),
        ],
    )


def kernel(head_embeds, tail_embeds, head_list, tail_list):
    out = _alpha_tc(head_embeds, tail_embeds)
    alpha = out.reshape(E)
    g_values = _gvalues_sc()(alpha, head_list)
    g_indices = jnp.stack([head_list, tail_list], axis=0)
    return (g_indices, g_values)
